# Initial kernel scaffold; baseline (speedup 1.0000x reference)
import jax
import jax.numpy as jnp
from jax.experimental import pallas as pl


def kernel(x, edge_index, mask_nodes, token_nodes, noise_nodes, noise_src, W1, b1, g1, be1, W2, b2, g2, be2, mask_token, We2d, Wd, bd):
    raise NotImplementedError("write your pallas kernel here")


# speedup vs baseline: 10.8339x; 10.8339x over previous
import functools

import jax
import jax.numpy as jnp
from jax import lax
from jax.experimental import pallas as pl
from jax.experimental.pallas import tpu as pltpu
from jax.experimental.pallas import tpu_sc as plsc

N = 10000
E = 320000
D = 128
EPS = 1e-5

NPAD = 10240
PAD_ROWS = NPAD - N
EPAD = 327680
EW = EPAD // 128
NC = 2
NS = 16
NW = NC * NS
EW_PER_W = EW // NW
ROWS_PER_TILE = NPAD // NS
MKW = 8192 // 128
TKW = 8192 // 128
NZW = 4096 // 128

_mesh = plsc.VectorSubcoreMesh(core_axis_name="c", subcore_axis_name="s",
                               num_cores=NC, num_subcores=NS)


def _fill_const(ref, val):
    v = jnp.full((16,), val, jnp.float32)
    for j in range(8):
        ref[pl.ds(j * 16, 16)] = v


def _fill_rows_zero(ref):
    z = jnp.zeros((16,), jnp.float32)

    def body(r, _):
        for j in range(8):
            ref[r, pl.ds(j * 16, 16)] = z
        return 0

    lax.fori_loop(0, 128, body, 0)


def _sc_setup_body(x_hbm, dstw, mknw, tknw, nnw, nsw,
                   deg_o, kd_o, tok_o, repl_o,
                   acc_deg, acc_kd, acc_tok, acc_repl,
                   idx_v, idx2_v, nn_v, ns_v, rows_v, ones_v, mones_v, zv_v):
    c = lax.axis_index("c")
    s = lax.axis_index("s")
    wid = c * NS + s
    base = s * ROWS_PER_TILE

    _fill_const(ones_v, 1.0)
    _fill_const(mones_v, -1.0)
    _fill_const(zv_v, 0.0)
    _fill_rows_zero(rows_v)

    for b in range(ROWS_PER_TILE // 128):
        off = base + b * 128
        pltpu.sync_copy(zv_v, acc_deg.at[pl.ds(off, 128)])
        pltpu.sync_copy(zv_v, acc_kd.at[pl.ds(off, 128)])
        pltpu.sync_copy(zv_v, acc_tok.at[pl.ds(off, 128)])
        pltpu.sync_copy(rows_v, acc_repl.at[pl.ds(off, 128), :])
    plsc.subcore_barrier()

    pltpu.sync_copy(dstw.at[pl.ds(wid * EW_PER_W, EW_PER_W), :], idx_v)

    def deg_step(j, _):
        pltpu.sync_copy(ones_v, acc_deg.at[idx_v.at[j]], add=True)
        return 0

    lax.fori_loop(0, EW_PER_W, deg_step, 0)

    pltpu.sync_copy(mknw.at[pl.ds(wid * (MKW // NW), MKW // NW), :], idx2_v)
    for j in range(MKW // NW):
        pltpu.sync_copy(mones_v, acc_kd.at[idx2_v.at[j]], add=True)

    pltpu.sync_copy(tknw.at[pl.ds(wid * (TKW // NW), TKW // NW), :], idx2_v)
    for j in range(TKW // NW):
        pltpu.sync_copy(ones_v, acc_tok.at[idx2_v.at[j]], add=True)

    pltpu.sync_copy(nnw.at[pl.ds(wid, 1), :], nn_v)
    pltpu.sync_copy(nsw.at[pl.ds(wid, 1), :], ns_v)
    pltpu.sync_copy(x_hbm.at[ns_v.at[0]], rows_v)
    pltpu.sync_copy(rows_v, acc_repl.at[nn_v.at[0]], add=True)

    plsc.subcore_barrier()
    for b in range(ROWS_PER_TILE // 128):
        off = base + b * 128
        pltpu.sync_copy(acc_deg.at[pl.ds(off, 128)], deg_o.at[c, pl.ds(off, 128)])
        pltpu.sync_copy(acc_kd.at[pl.ds(off, 128)], kd_o.at[c, pl.ds(off, 128)])
        pltpu.sync_copy(acc_tok.at[pl.ds(off, 128)], tok_o.at[c, pl.ds(off, 128)])
        pltpu.sync_copy(acc_repl.at[pl.ds(off, 128), :],
                        repl_o.at[c, pl.ds(off, 128), :])


_sc_setup = pl.kernel(
    _sc_setup_body,
    out_type=(
        jax.ShapeDtypeStruct((NC, NPAD), jnp.float32),
        jax.ShapeDtypeStruct((NC, NPAD), jnp.float32),
        jax.ShapeDtypeStruct((NC, NPAD), jnp.float32),
        jax.ShapeDtypeStruct((NC, NPAD, D), jnp.float32),
    ),
    mesh=_mesh,
    scratch_types=[
        pltpu.VMEM_SHARED((NPAD,), jnp.float32),
        pltpu.VMEM_SHARED((NPAD,), jnp.float32),
        pltpu.VMEM_SHARED((NPAD,), jnp.float32),
        pltpu.VMEM_SHARED((NPAD, D), jnp.float32),
        pltpu.VMEM((EW_PER_W, 128), jnp.int32),
        pltpu.VMEM((MKW // NW, 128), jnp.int32),
        pltpu.VMEM((1, 128), jnp.int32),
        pltpu.VMEM((1, 128), jnp.int32),
        pltpu.VMEM((128, D), jnp.float32),
        pltpu.VMEM((128,), jnp.float32),
        pltpu.VMEM((128,), jnp.float32),
        pltpu.VMEM((128,), jnp.float32),
    ],
    name="sc_graph_setup",
)


def _sc_agg_body(hs_hbm, srcw, dstw, out_o, acc, src_v, dst_v, rows_v):
    c = lax.axis_index("c")
    s = lax.axis_index("s")
    wid = c * NS + s
    base = s * ROWS_PER_TILE

    _fill_rows_zero(rows_v)
    for b in range(ROWS_PER_TILE // 128):
        pltpu.sync_copy(rows_v, acc.at[pl.ds(base + b * 128, 128), :])
    plsc.subcore_barrier()

    ebase = wid * EW_PER_W
    pltpu.sync_copy(srcw.at[pl.ds(ebase, EW_PER_W), :], src_v)
    pltpu.sync_copy(dstw.at[pl.ds(ebase, EW_PER_W), :], dst_v)

    def step(j, _):
        pltpu.sync_copy(hs_hbm.at[src_v.at[j]], rows_v)
        pltpu.sync_copy(rows_v, acc.at[dst_v.at[j]], add=True)
        return 0

    lax.fori_loop(0, EW_PER_W, step, 0)

    plsc.subcore_barrier()
    for b in range(ROWS_PER_TILE // 128):
        off = base + b * 128
        pltpu.sync_copy(acc.at[pl.ds(off, 128), :], out_o.at[c, pl.ds(off, 128), :])


_sc_agg = pl.kernel(
    _sc_agg_body,
    out_type=jax.ShapeDtypeStruct((NC, NPAD, D), jnp.float32),
    mesh=_mesh,
    scratch_types=[
        pltpu.VMEM_SHARED((NPAD, D), jnp.float32),
        pltpu.VMEM((EW_PER_W, 128), jnp.int32),
        pltpu.VMEM((EW_PER_W, 128), jnp.int32),
        pltpu.VMEM((128, D), jnp.float32),
    ],
    name="sc_gcn_agg",
)


BR = 1024
GRID = NPAD // BR

_row_spec = pl.BlockSpec((BR, D), lambda i: (i, 0))
_w_spec = pl.BlockSpec((D, D), lambda i: (0, 0))
_v_spec = pl.BlockSpec((1, D), lambda i: (0, 0))


def _tc_mask_body(x, r0, r1, keepb, tokb, dinvb, mt, h0_o, hs0_o):
    h0 = x[...] * keepb[...] + r0[...] + r1[...] + tokb[...] * mt[...]
    h0_o[...] = h0
    hs0_o[...] = h0 * dinvb[...]


def _tc_layer_body(a0, a1, hs, hprev, dinvb, W, b, g, be, h_o, hs_o):
    dinv = dinvb[...]
    agg = (a0[...] + a1[...] + hs[...]) * dinv
    t = jnp.dot(agg, W[...], preferred_element_type=jnp.float32) + b[...] + hprev[...]
    mu = jnp.mean(t, axis=-1, keepdims=True)
    var = jnp.mean((t - mu) * (t - mu), axis=-1, keepdims=True)
    y = (t - mu) * lax.rsqrt(var + EPS) * g[...] + be[...]
    h = jnp.maximum(y, 0.0)
    h_o[...] = h
    hs_o[...] = h * dinv


def _tc_e2d_body(a0, a1, hs, hprev, dinvb, keepb, W, b, g, be, We2d, hs2_o):
    dinv = dinvb[...]
    agg = (a0[...] + a1[...] + hs[...]) * dinv
    t = jnp.dot(agg, W[...], preferred_element_type=jnp.float32) + b[...] + hprev[...]
    mu = jnp.mean(t, axis=-1, keepdims=True)
    var = jnp.mean((t - mu) * (t - mu), axis=-1, keepdims=True)
    y = (t - mu) * lax.rsqrt(var + EPS) * g[...] + be[...]
    h2 = jnp.maximum(y, 0.0)
    rep = jnp.dot(h2, We2d[...], preferred_element_type=jnp.float32) * keepb[...]
    hs2_o[...] = rep * dinv


def _tc_loss_body(a0, a1, hs2, dinvb, keepb, x, Wd, bd, out):
    agg = (a0[...] + a1[...] + hs2[...]) * dinvb[...]
    recon = jnp.dot(agg, Wd[...], preferred_element_type=jnp.float32) + bd[...]
    xv = x[...]
    xn = xv / (jnp.sqrt(jnp.sum(xv * xv, axis=-1, keepdims=True)) + 1e-8)
    yn = recon / (jnp.sqrt(jnp.sum(recon * recon, axis=-1, keepdims=True)) + 1e-8)
    cos = jnp.sum(xn * yn, axis=-1, keepdims=True)
    d = 1.0 - cos
    li = d * d
    pid = pl.program_id(0)

    @pl.when(pid == 0)
    def _():
        out[...] = jnp.zeros((1, D), jnp.float32)

    rid = lax.broadcasted_iota(jnp.int32, (BR, 1), 0) + pid * BR
    w = jnp.where(rid < N, 1.0 - keepb[:, :1], 0.0)
    out[...] += jnp.broadcast_to(jnp.sum(li * w), (1, D))


_tc_mask = pl.pallas_call(
    _tc_mask_body,
    grid=(GRID,),
    in_specs=[_row_spec] * 6 + [_v_spec],
    out_specs=[_row_spec, _row_spec],
    out_shape=[jax.ShapeDtypeStruct((NPAD, D), jnp.float32)] * 2,
    name="tc_mask",
)

_tc_layer = pl.pallas_call(
    _tc_layer_body,
    grid=(GRID,),
    in_specs=[_row_spec] * 5 + [_w_spec] + [_v_spec] * 3,
    out_specs=[_row_spec, _row_spec],
    out_shape=[jax.ShapeDtypeStruct((NPAD, D), jnp.float32)] * 2,
    name="tc_gcn_layer",
)

_tc_e2d = pl.pallas_call(
    _tc_e2d_body,
    grid=(GRID,),
    in_specs=[_row_spec] * 6 + [_w_spec] + [_v_spec] * 3 + [_w_spec],
    out_specs=_row_spec,
    out_shape=jax.ShapeDtypeStruct((NPAD, D), jnp.float32),
    name="tc_layer_e2d",
)

_tc_loss = pl.pallas_call(
    _tc_loss_body,
    grid=(GRID,),
    in_specs=[_row_spec] * 6 + [_w_spec] + [_v_spec],
    out_specs=pl.BlockSpec((1, D), lambda i: (0, 0)),
    out_shape=jax.ShapeDtypeStruct((1, D), jnp.float32),
    name="tc_recon_loss",
)


def _pad_idx_windows(a, total):
    k = total - a.shape[0]
    p = N + (jnp.arange(k, dtype=jnp.int32) % PAD_ROWS)
    return jnp.concatenate([a.astype(jnp.int32), p]).reshape(total // 128, 128)


@jax.jit
def kernel(x, edge_index, mask_nodes, token_nodes, noise_nodes, noise_src,
           W1, b1, g1, be1, W2, b2, g2, be2, mask_token, We2d, Wd, bd):
    num_mask = mask_nodes.shape[0]
    srcw = _pad_idx_windows(edge_index[0], EPAD)
    dstw = _pad_idx_windows(edge_index[1], EPAD)
    mknw = _pad_idx_windows(mask_nodes, MKW * 128)
    tknw = _pad_idx_windows(token_nodes, TKW * 128)
    nnw = _pad_idx_windows(noise_nodes, NZW * 128)
    nsw = _pad_idx_windows(noise_src, NZW * 128)
    x_p = jnp.zeros((NPAD, D), jnp.float32).at[:N].set(x)

    deg_p, kd_p, tok_p, repl_p = _sc_setup(x_p, dstw, mknw, tknw, nnw, nsw)

    dinv = lax.rsqrt(deg_p[0] + deg_p[1] + 1.0)
    keep = 1.0 + kd_p[0] + kd_p[1]
    tok = tok_p[0] + tok_p[1]
    dinvb = jnp.broadcast_to(dinv[:, None], (NPAD, D))
    keepb = jnp.broadcast_to(keep[:, None], (NPAD, D))
    tokb = jnp.broadcast_to(tok[:, None], (NPAD, D))

    h0, hs0 = _tc_mask(x_p, repl_p[0], repl_p[1], keepb, tokb, dinvb,
                       mask_token.reshape(1, D))

    a = _sc_agg(hs0, srcw, dstw)
    h1, hs1 = _tc_layer(a[0], a[1], hs0, h0, dinvb,
                        W1, b1.reshape(1, D), g1.reshape(1, D), be1.reshape(1, D))

    a = _sc_agg(hs1, srcw, dstw)
    hs2 = _tc_e2d(a[0], a[1], hs1, h1, dinvb, keepb,
                  W2, b2.reshape(1, D), g2.reshape(1, D), be2.reshape(1, D), We2d)

    a = _sc_agg(hs2, srcw, dstw)
    parts = _tc_loss(a[0], a[1], hs2, dinvb, keepb, x_p, Wd, bd.reshape(1, D))
    return parts[0, 0] / num_mask


# speedup vs baseline: 15.6549x; 1.4450x over previous
import functools

import jax
import jax.numpy as jnp
from jax import lax
from jax.experimental import pallas as pl
from jax.experimental.pallas import tpu as pltpu
from jax.experimental.pallas import tpu_sc as plsc

N = 10000
E = 320000
D = 128
EPS = 1e-5

NPAD = 10240
PAD_ROWS = NPAD - N
EPAD = 327680
EW = EPAD // 128
NC = 2
NS = 16
NW = NC * NS
EW_PER_W = EW // NW
ROWS_PER_TILE = NPAD // NS
MKW = 8192 // 128
TKW = 8192 // 128
NZW = 4096 // 128
NB = 2
IDXC = 16

_mesh = plsc.VectorSubcoreMesh(core_axis_name="c", subcore_axis_name="s",
                               num_cores=NC, num_subcores=NS)


def _fill_const(ref, val):
    v = jnp.full((16,), val, jnp.float32)
    for j in range(8):
        ref[pl.ds(j * 16, 16)] = v


def _fill_rows_zero(ref):
    z = jnp.zeros((16,), jnp.float32)

    def body(r, _):
        for j in range(8):
            ref[r, pl.ds(j * 16, 16)] = z
        return 0

    lax.fori_loop(0, 128, body, 0)


def _sc_setup_body(x_hbm, dstw, mknw, tknw, nnw, nsw,
                   deg_o, kd_o, tok_o, repl_o,
                   acc_deg, acc_kd, acc_tok, acc_repl,
                   idx_v, idx2_v, nn_v, ns_v, rows_v, ones_v, mones_v, zv_v):
    c = lax.axis_index("c")
    s = lax.axis_index("s")
    wid = c * NS + s
    base = s * ROWS_PER_TILE

    _fill_const(ones_v, 1.0)
    _fill_const(mones_v, -1.0)
    _fill_const(zv_v, 0.0)
    _fill_rows_zero(rows_v)

    for b in range(ROWS_PER_TILE // 128):
        off = base + b * 128
        pltpu.sync_copy(zv_v, acc_deg.at[pl.ds(off, 128)])
        pltpu.sync_copy(zv_v, acc_kd.at[pl.ds(off, 128)])
        pltpu.sync_copy(zv_v, acc_tok.at[pl.ds(off, 128)])
        pltpu.sync_copy(rows_v, acc_repl.at[pl.ds(off, 128), :])
    plsc.subcore_barrier()

    pltpu.sync_copy(dstw.at[pl.ds(wid * EW_PER_W, EW_PER_W), :], idx_v)

    def deg_step(j, _):
        pltpu.sync_copy(ones_v, acc_deg.at[idx_v.at[j]], add=True)
        return 0

    lax.fori_loop(0, EW_PER_W, deg_step, 0)

    pltpu.sync_copy(mknw.at[pl.ds(wid * (MKW // NW), MKW // NW), :], idx2_v)
    for j in range(MKW // NW):
        pltpu.sync_copy(mones_v, acc_kd.at[idx2_v.at[j]], add=True)

    pltpu.sync_copy(tknw.at[pl.ds(wid * (TKW // NW), TKW // NW), :], idx2_v)
    for j in range(TKW // NW):
        pltpu.sync_copy(ones_v, acc_tok.at[idx2_v.at[j]], add=True)

    pltpu.sync_copy(nnw.at[pl.ds(wid, 1), :], nn_v)
    pltpu.sync_copy(nsw.at[pl.ds(wid, 1), :], ns_v)
    pltpu.sync_copy(x_hbm.at[ns_v.at[0]], rows_v)
    pltpu.sync_copy(rows_v, acc_repl.at[nn_v.at[0]], add=True)

    plsc.subcore_barrier()
    for b in range(ROWS_PER_TILE // 128):
        off = base + b * 128
        pltpu.sync_copy(acc_deg.at[pl.ds(off, 128)], deg_o.at[c, pl.ds(off, 128)])
        pltpu.sync_copy(acc_kd.at[pl.ds(off, 128)], kd_o.at[c, pl.ds(off, 128)])
        pltpu.sync_copy(acc_tok.at[pl.ds(off, 128)], tok_o.at[c, pl.ds(off, 128)])
        pltpu.sync_copy(acc_repl.at[pl.ds(off, 128), :],
                        repl_o.at[c, pl.ds(off, 128), :])


_sc_setup = pl.kernel(
    _sc_setup_body,
    out_type=(
        jax.ShapeDtypeStruct((NC, NPAD), jnp.float32),
        jax.ShapeDtypeStruct((NC, NPAD), jnp.float32),
        jax.ShapeDtypeStruct((NC, NPAD), jnp.float32),
        jax.ShapeDtypeStruct((NC, NPAD, D), jnp.float32),
    ),
    mesh=_mesh,
    scratch_types=[
        pltpu.VMEM_SHARED((NPAD,), jnp.float32),
        pltpu.VMEM_SHARED((NPAD,), jnp.float32),
        pltpu.VMEM_SHARED((NPAD,), jnp.float32),
        pltpu.VMEM_SHARED((NPAD, D), jnp.float32),
        pltpu.VMEM((EW_PER_W, 128), jnp.int32),
        pltpu.VMEM((MKW // NW, 128), jnp.int32),
        pltpu.VMEM((1, 128), jnp.int32),
        pltpu.VMEM((1, 128), jnp.int32),
        pltpu.VMEM((128, D), jnp.float32),
        pltpu.VMEM((128,), jnp.float32),
        pltpu.VMEM((128,), jnp.float32),
        pltpu.VMEM((128,), jnp.float32),
    ],
    name="sc_graph_setup",
)


def _sc_agg_body(hs_hbm, srcw, dstw, out_o, acc, src_v, dst_v, rows_v, s0, s1):
    c = lax.axis_index("c")
    s = lax.axis_index("s")
    wid = c * NS + s
    base = s * ROWS_PER_TILE
    sems = [s0, s1]

    z = jnp.zeros((16,), jnp.float32)

    def zbody(r, _):
        for j in range(8):
            rows_v[0, r, pl.ds(j * 16, 16)] = z
        return 0

    lax.fori_loop(0, 128, zbody, 0)
    for b in range(ROWS_PER_TILE // 128):
        pltpu.sync_copy(rows_v.at[0], acc.at[pl.ds(base + b * 128, 128), :])
    plsc.subcore_barrier()

    ebase = wid * EW_PER_W

    def gstart(j, b):
        pltpu.async_copy(hs_hbm.at[src_v.at[j]], rows_v.at[b], sems[b])

    def gwait(b):
        pltpu.make_async_copy(hs_hbm.at[src_v.at[0]], rows_v.at[b],
                              sems[b]).wait()

    def chunk(o, _):
        pltpu.sync_copy(srcw.at[pl.ds(ebase + o * IDXC, IDXC), :], src_v)
        pltpu.sync_copy(dstw.at[pl.ds(ebase + o * IDXC, IDXC), :], dst_v)
        gstart(0, 0)
        gstart(1, 1)
        for j in range(IDXC):
            b = j % NB
            gwait(b)
            pltpu.sync_copy(rows_v.at[b], acc.at[dst_v.at[j]], add=True)
            if j + NB < IDXC:
                gstart(j + NB, b)
        return 0

    lax.fori_loop(0, EW_PER_W // IDXC, chunk, 0)

    plsc.subcore_barrier()
    for b in range(ROWS_PER_TILE // 128):
        off = base + b * 128
        pltpu.sync_copy(acc.at[pl.ds(off, 128), :], out_o.at[c, pl.ds(off, 128), :])


_sc_agg = pl.kernel(
    _sc_agg_body,
    out_type=jax.ShapeDtypeStruct((NC, NPAD, D), jnp.float32),
    mesh=_mesh,
    scratch_types=[
        pltpu.VMEM_SHARED((NPAD, D), jnp.float32),
        pltpu.VMEM((IDXC, 128), jnp.int32),
        pltpu.VMEM((IDXC, 128), jnp.int32),
        pltpu.VMEM((NB, 128, D), jnp.float32),
        pltpu.SemaphoreType.DMA,
        pltpu.SemaphoreType.DMA,
    ],
    name="sc_gcn_agg",
)


BR = 2000
GRID = N // BR

_row_spec = pl.BlockSpec((BR, D), lambda i: (i, 0))
_col_spec = pl.BlockSpec((BR, 1), lambda i: (i, 0))
_pp_spec = pl.BlockSpec((NC, BR, D), lambda i: (0, i, 0))
_w_spec = pl.BlockSpec((D, D), lambda i: (0, 0))
_v_spec = pl.BlockSpec((1, D), lambda i: (0, 0))


def _tc_mask_body(x, rp, degc, kdc, tokc, mt, h0_o, hs0_o):
    keep = 1.0 + kdc[...]
    dinv = lax.rsqrt(degc[...] + 1.0)
    r = rp[...]
    h0 = x[...] * keep + r[0] + r[1] + tokc[...] * mt[...]
    h0_o[...] = h0
    hs0_o[...] = h0 * dinv


def _tc_layer_body(ap, hs, hprev, degc, W, b, g, be, h_o, hs_o):
    dinv = lax.rsqrt(degc[...] + 1.0)
    a = ap[...]
    agg = (a[0] + a[1] + hs[...]) * dinv
    t = jnp.dot(agg, W[...], preferred_element_type=jnp.float32) + b[...] + hprev[...]
    mu = jnp.mean(t, axis=-1, keepdims=True)
    var = jnp.mean((t - mu) * (t - mu), axis=-1, keepdims=True)
    y = (t - mu) * lax.rsqrt(var + EPS) * g[...] + be[...]
    h = jnp.maximum(y, 0.0)
    h_o[...] = h
    hs_o[...] = h * dinv


def _tc_e2d_body(ap, hs, hprev, degc, kdc, W, b, g, be, We2d, hs2_o):
    dinv = lax.rsqrt(degc[...] + 1.0)
    a = ap[...]
    agg = (a[0] + a[1] + hs[...]) * dinv
    t = jnp.dot(agg, W[...], preferred_element_type=jnp.float32) + b[...] + hprev[...]
    mu = jnp.mean(t, axis=-1, keepdims=True)
    var = jnp.mean((t - mu) * (t - mu), axis=-1, keepdims=True)
    y = (t - mu) * lax.rsqrt(var + EPS) * g[...] + be[...]
    h2 = jnp.maximum(y, 0.0)
    rep = jnp.dot(h2, We2d[...], preferred_element_type=jnp.float32) * (1.0 + kdc[...])
    hs2_o[...] = rep * dinv


def _tc_loss_body(ap, hs2, degc, kdc, x, Wd, bd, out):
    dinv = lax.rsqrt(degc[...] + 1.0)
    a = ap[...]
    agg = (a[0] + a[1] + hs2[...]) * dinv
    recon = jnp.dot(agg, Wd[...], preferred_element_type=jnp.float32) + bd[...]
    xv = x[...]
    xn = xv / (jnp.sqrt(jnp.sum(xv * xv, axis=-1, keepdims=True)) + 1e-8)
    yn = recon / (jnp.sqrt(jnp.sum(recon * recon, axis=-1, keepdims=True)) + 1e-8)
    cos = jnp.sum(xn * yn, axis=-1, keepdims=True)
    d = 1.0 - cos
    li = d * d
    pid = pl.program_id(0)

    @pl.when(pid == 0)
    def _():
        out[...] = jnp.zeros((1, D), jnp.float32)

    w = -kdc[...]
    out[...] += jnp.broadcast_to(jnp.sum(li * w), (1, D))


_tc_mask = pl.pallas_call(
    _tc_mask_body,
    grid=(GRID,),
    in_specs=[_row_spec, _pp_spec] + [_col_spec] * 3 + [_v_spec],
    out_specs=[_row_spec, _row_spec],
    out_shape=[jax.ShapeDtypeStruct((N, D), jnp.float32)] * 2,
    name="tc_mask",
)

_tc_layer = pl.pallas_call(
    _tc_layer_body,
    grid=(GRID,),
    in_specs=[_pp_spec] + [_row_spec] * 2 + [_col_spec] + [_w_spec] + [_v_spec] * 3,
    out_specs=[_row_spec, _row_spec],
    out_shape=[jax.ShapeDtypeStruct((N, D), jnp.float32)] * 2,
    name="tc_gcn_layer",
)

_tc_e2d = pl.pallas_call(
    _tc_e2d_body,
    grid=(GRID,),
    in_specs=[_pp_spec] + [_row_spec] * 2 + [_col_spec] * 2 + [_w_spec] + [_v_spec] * 3 + [_w_spec],
    out_specs=_row_spec,
    out_shape=jax.ShapeDtypeStruct((N, D), jnp.float32),
    name="tc_layer_e2d",
)

_tc_loss = pl.pallas_call(
    _tc_loss_body,
    grid=(GRID,),
    in_specs=[_pp_spec] + [_row_spec] + [_col_spec] * 2 + [_row_spec] + [_w_spec] + [_v_spec],
    out_specs=pl.BlockSpec((1, D), lambda i: (0, 0)),
    out_shape=jax.ShapeDtypeStruct((1, D), jnp.float32),
    name="tc_recon_loss",
)


def _pad_idx_windows(a, total, pad_real_rows):
    k = total - a.shape[0]
    p = (0 if pad_real_rows else N) + (jnp.arange(k, dtype=jnp.int32) % PAD_ROWS)
    return jnp.concatenate([a.astype(jnp.int32), p]).reshape(total // 128, 128)


@jax.jit
def kernel(x, edge_index, mask_nodes, token_nodes, noise_nodes, noise_src,
           W1, b1, g1, be1, W2, b2, g2, be2, mask_token, We2d, Wd, bd):
    num_mask = mask_nodes.shape[0]
    srcw = _pad_idx_windows(edge_index[0], EPAD, True)
    dstw = _pad_idx_windows(edge_index[1], EPAD, False)
    mknw = _pad_idx_windows(mask_nodes, MKW * 128, False)
    tknw = _pad_idx_windows(token_nodes, TKW * 128, False)
    nnw = _pad_idx_windows(noise_nodes, NZW * 128, False)
    nsw = _pad_idx_windows(noise_src, NZW * 128, True)

    deg_p, kd_p, tok_p, repl_p = _sc_setup(x, dstw, mknw, tknw, nnw, nsw)

    degc = (deg_p[0, :N] + deg_p[1, :N])[:, None]
    kdc = (kd_p[0, :N] + kd_p[1, :N])[:, None]
    tokc = (tok_p[0, :N] + tok_p[1, :N])[:, None]

    h0, hs0 = _tc_mask(x, repl_p, degc, kdc, tokc, mask_token.reshape(1, D))

    a = _sc_agg(hs0, srcw, dstw)
    h1, hs1 = _tc_layer(a, hs0, h0, degc,
                        W1, b1.reshape(1, D), g1.reshape(1, D), be1.reshape(1, D))

    a = _sc_agg(hs1, srcw, dstw)
    hs2 = _tc_e2d(a, hs1, h1, degc, kdc,
                  W2, b2.reshape(1, D), g2.reshape(1, D), be2.reshape(1, D), We2d)

    a = _sc_agg(hs2, srcw, dstw)
    parts = _tc_loss(a, hs2, degc, kdc, x, Wd, bd.reshape(1, D))
    return parts[0, 0] / num_mask


# speedup vs baseline: 16.0290x; 1.0239x over previous
import functools

import jax
import jax.numpy as jnp
import numpy as np
from jax import lax
from jax.experimental import pallas as pl
from jax.experimental.pallas import tpu as pltpu
from jax.experimental.pallas import tpu_sc as plsc

N = 10000
E = 320000
D = 128
EPS = 1e-5

NPAD = 10240
PAD_ROWS = NPAD - N
EWIN = 128
EPAD = 327680
EW = EPAD // EWIN
NC = 2
NS = 16
NW = NC * NS
EW_PER_W = EW // NW
ROWS_PER_TILE = NPAD // NS
MKW = 8192 // EWIN
TKW = 8192 // EWIN
NZW = 4096 // EWIN
NB = 2
IDXC = 16

_mesh = plsc.VectorSubcoreMesh(core_axis_name="c", subcore_axis_name="s",
                               num_cores=NC, num_subcores=NS)


def _fill_const(ref, val):
    v = jnp.full((16,), val, jnp.float32)
    for j in range(EWIN // 16):
        ref[pl.ds(j * 16, 16)] = v


def _fill_rows_zero(ref):
    z = jnp.zeros((16,), jnp.float32)

    def body(r, _):
        for j in range(D // 16):
            ref[r, pl.ds(j * 16, 16)] = z
        return 0

    lax.fori_loop(0, EWIN, body, 0)


def _sc_setup_body(x_hbm, dstw, mknw, tknw, nnw, nsw,
                   deg_o, kd_o, tok_o, repl_o,
                   acc_deg, acc_kd, acc_tok, acc_repl,
                   idx_v, idx2_v, nn_v, ns_v, rows_v, ones_v, mones_v, zv_v,
                   ds0, ds1, ds2, ds3):
    c = lax.axis_index("c")
    s = lax.axis_index("s")
    wid = c * NS + s
    base = s * ROWS_PER_TILE

    _fill_const(ones_v, 1.0)
    _fill_const(mones_v, -1.0)
    _fill_const(zv_v, 0.0)
    _fill_rows_zero(rows_v)

    for b in range(ROWS_PER_TILE // EWIN):
        off = base + b * EWIN
        pltpu.sync_copy(zv_v, acc_deg.at[pl.ds(off, EWIN)])
        pltpu.sync_copy(zv_v, acc_kd.at[pl.ds(off, EWIN)])
        pltpu.sync_copy(zv_v, acc_tok.at[pl.ds(off, EWIN)])
        pltpu.sync_copy(rows_v, acc_repl.at[pl.ds(off, EWIN), :])
    plsc.subcore_barrier()

    pltpu.sync_copy(dstw.at[pl.ds(wid * EW_PER_W, EW_PER_W), :], idx_v)
    dsems = [ds0, ds1, ds2, ds3]

    def dstart(j, b):
        pltpu.async_copy(ones_v, acc_deg.at[idx_v.at[j]], dsems[b], add=True)

    def dwait(b):
        pltpu.make_async_copy(ones_v, acc_deg.at[idx_v.at[0]], dsems[b]).wait()

    for b in range(4):
        dstart(b, b)

    def deg_step(o, _):
        for b in range(4):
            j = o * 4 + b
            dwait(b)
            dstart(j + 4, b)
        return 0

    lax.fori_loop(0, EW_PER_W // 4 - 1, deg_step, 0)
    for b in range(4):
        dwait(b)

    pltpu.sync_copy(mknw.at[pl.ds(wid * (MKW // NW), MKW // NW), :], idx2_v)
    for j in range(MKW // NW):
        pltpu.sync_copy(mones_v, acc_kd.at[idx2_v.at[j]], add=True)

    pltpu.sync_copy(tknw.at[pl.ds(wid * (TKW // NW), TKW // NW), :], idx2_v)
    for j in range(TKW // NW):
        pltpu.sync_copy(ones_v, acc_tok.at[idx2_v.at[j]], add=True)

    pltpu.sync_copy(nnw.at[pl.ds(wid, 1), :], nn_v)
    pltpu.sync_copy(nsw.at[pl.ds(wid, 1), :], ns_v)
    pltpu.sync_copy(x_hbm.at[ns_v.at[0]], rows_v)
    pltpu.sync_copy(rows_v, acc_repl.at[nn_v.at[0]], add=True)

    plsc.subcore_barrier()
    pltpu.sync_copy(acc_deg.at[pl.ds(base, ROWS_PER_TILE)],
                    deg_o.at[c, pl.ds(base, ROWS_PER_TILE)])
    pltpu.sync_copy(acc_kd.at[pl.ds(base, ROWS_PER_TILE)],
                    kd_o.at[c, pl.ds(base, ROWS_PER_TILE)])
    pltpu.sync_copy(acc_tok.at[pl.ds(base, ROWS_PER_TILE)],
                    tok_o.at[c, pl.ds(base, ROWS_PER_TILE)])
    pltpu.sync_copy(acc_repl.at[pl.ds(base, ROWS_PER_TILE), :],
                    repl_o.at[c, pl.ds(base, ROWS_PER_TILE), :])


_sc_setup = pl.kernel(
    _sc_setup_body,
    out_type=(
        jax.ShapeDtypeStruct((NC, NPAD), jnp.float32),
        jax.ShapeDtypeStruct((NC, NPAD), jnp.float32),
        jax.ShapeDtypeStruct((NC, NPAD), jnp.float32),
        jax.ShapeDtypeStruct((NC, NPAD, D), jnp.float32),
    ),
    mesh=_mesh,
    scratch_types=[
        pltpu.VMEM_SHARED((NPAD,), jnp.float32),
        pltpu.VMEM_SHARED((NPAD,), jnp.float32),
        pltpu.VMEM_SHARED((NPAD,), jnp.float32),
        pltpu.VMEM_SHARED((NPAD, D), jnp.float32),
        pltpu.VMEM((EW_PER_W, EWIN), jnp.int32),
        pltpu.VMEM((MKW // NW, EWIN), jnp.int32),
        pltpu.VMEM((1, EWIN), jnp.int32),
        pltpu.VMEM((1, EWIN), jnp.int32),
        pltpu.VMEM((EWIN, D), jnp.float32),
        pltpu.VMEM((EWIN,), jnp.float32),
        pltpu.VMEM((EWIN,), jnp.float32),
        pltpu.VMEM((EWIN,), jnp.float32),
        pltpu.SemaphoreType.DMA,
        pltpu.SemaphoreType.DMA,
        pltpu.SemaphoreType.DMA,
        pltpu.SemaphoreType.DMA,
    ],
    name="sc_graph_setup",
)


def _sc_agg_body(hs_hbm, srcw, dstw, out_o, acc, src_v, dst_v, rows_v,
                 s0, s1, s2, s3):
    c = lax.axis_index("c")
    s = lax.axis_index("s")
    wid = c * NS + s
    base = s * ROWS_PER_TILE
    gsems = [s0, s1, s2, s3]

    z = jnp.zeros((16,), jnp.float32)

    def zbody(r, _):
        for j in range(D // 16):
            rows_v[0, r, pl.ds(j * 16, 16)] = z
        return 0

    lax.fori_loop(0, EWIN, zbody, 0)
    for b in range(ROWS_PER_TILE // EWIN):
        pltpu.sync_copy(rows_v.at[0], acc.at[pl.ds(base + b * EWIN, EWIN), :])
    plsc.subcore_barrier()

    ebase = wid * EW_PER_W

    def gstart(j, b):
        pltpu.async_copy(hs_hbm.at[src_v.at[j]], rows_v.at[b], gsems[b])

    def gwait(b):
        pltpu.make_async_copy(hs_hbm.at[src_v.at[0]], rows_v.at[b],
                              gsems[b]).wait()

    def chunk(o, _):
        pltpu.sync_copy(srcw.at[pl.ds(ebase + o * IDXC, IDXC), :], src_v)
        pltpu.sync_copy(dstw.at[pl.ds(ebase + o * IDXC, IDXC), :], dst_v)
        for b in range(NB):
            gstart(b, b)
        for j in range(IDXC):
            b = j % NB
            gwait(b)
            pltpu.sync_copy(rows_v.at[b], acc.at[dst_v.at[j]], add=True)
            if j + NB < IDXC:
                gstart(j + NB, b)
        return 0

    lax.fori_loop(0, EW_PER_W // IDXC, chunk, 0)

    plsc.subcore_barrier()
    pltpu.sync_copy(acc.at[pl.ds(base, ROWS_PER_TILE), :],
                    out_o.at[c, pl.ds(base, ROWS_PER_TILE), :])


_sc_agg = pl.kernel(
    _sc_agg_body,
    out_type=jax.ShapeDtypeStruct((NC, NPAD, D), jnp.float32),
    mesh=_mesh,
    scratch_types=[
        pltpu.VMEM_SHARED((NPAD, D), jnp.float32),
        pltpu.VMEM((IDXC, EWIN), jnp.int32),
        pltpu.VMEM((IDXC, EWIN), jnp.int32),
        pltpu.VMEM((NB, EWIN, D), jnp.float32),
        pltpu.SemaphoreType.DMA,
        pltpu.SemaphoreType.DMA,
        pltpu.SemaphoreType.DMA,
        pltpu.SemaphoreType.DMA,
    ],
    name="sc_gcn_agg",
)


BR = 2000
GRID = N // BR

_row_spec = pl.BlockSpec((BR, D), lambda i: (i, 0))
_col_spec = pl.BlockSpec((BR, 1), lambda i: (i, 0))
_pp_spec = pl.BlockSpec((NC, BR, D), lambda i: (0, i, 0))
_w_spec = pl.BlockSpec((D, D), lambda i: (0, 0))
_v_spec = pl.BlockSpec((1, D), lambda i: (0, 0))


def _tc_mask_body(x, rp, degc, kdc, tokc, mt, hs0_o):
    keep = 1.0 + kdc[...]
    dinv = lax.rsqrt(degc[...] + 1.0)
    r = rp[...]
    h0 = x[...] * keep + r[0] + r[1] + tokc[...] * mt[...]
    hs0_o[...] = h0 * dinv


def _tc_layer_body(ap, hs, degc, W, b, g, be, hs_o):
    deg = degc[...] + 1.0
    dinv = lax.rsqrt(deg)
    hsv = hs[...]
    hprev = hsv * jnp.sqrt(deg)
    a = ap[...]
    agg = (a[0] + a[1] + hsv) * dinv
    t = jnp.dot(agg, W[...], preferred_element_type=jnp.float32) + b[...] + hprev
    mu = jnp.mean(t, axis=-1, keepdims=True)
    var = jnp.mean((t - mu) * (t - mu), axis=-1, keepdims=True)
    y = (t - mu) * lax.rsqrt(var + EPS) * g[...] + be[...]
    h = jnp.maximum(y, 0.0)
    hs_o[...] = h * dinv


def _tc_e2d_body(ap, hs, degc, kdc, W, b, g, be, We2d, hs2_o):
    deg = degc[...] + 1.0
    dinv = lax.rsqrt(deg)
    hsv = hs[...]
    hprev = hsv * jnp.sqrt(deg)
    a = ap[...]
    agg = (a[0] + a[1] + hsv) * dinv
    t = jnp.dot(agg, W[...], preferred_element_type=jnp.float32) + b[...] + hprev
    mu = jnp.mean(t, axis=-1, keepdims=True)
    var = jnp.mean((t - mu) * (t - mu), axis=-1, keepdims=True)
    y = (t - mu) * lax.rsqrt(var + EPS) * g[...] + be[...]
    h2 = jnp.maximum(y, 0.0)
    rep = jnp.dot(h2, We2d[...], preferred_element_type=jnp.float32) * (1.0 + kdc[...])
    hs2_o[...] = rep * dinv


def _tc_loss_body(ap, hs2, degc, kdc, x, Wd, bd, out):
    dinv = lax.rsqrt(degc[...] + 1.0)
    a = ap[...]
    agg = (a[0] + a[1] + hs2[...]) * dinv
    recon = jnp.dot(agg, Wd[...], preferred_element_type=jnp.float32) + bd[...]
    xv = x[...]
    xn = xv / (jnp.sqrt(jnp.sum(xv * xv, axis=-1, keepdims=True)) + 1e-8)
    yn = recon / (jnp.sqrt(jnp.sum(recon * recon, axis=-1, keepdims=True)) + 1e-8)
    cos = jnp.sum(xn * yn, axis=-1, keepdims=True)
    d = 1.0 - cos
    li = d * d
    pid = pl.program_id(0)

    @pl.when(pid == 0)
    def _():
        out[...] = jnp.zeros((1, D), jnp.float32)

    w = -kdc[...]
    out[...] += jnp.broadcast_to(jnp.sum(li * w), (1, D))


_tc_mask = pl.pallas_call(
    _tc_mask_body,
    grid=(GRID,),
    in_specs=[_row_spec, _pp_spec] + [_col_spec] * 3 + [_v_spec],
    out_specs=_row_spec,
    out_shape=jax.ShapeDtypeStruct((N, D), jnp.float32),
    name="tc_mask",
)

_tc_layer = pl.pallas_call(
    _tc_layer_body,
    grid=(GRID,),
    in_specs=[_pp_spec, _row_spec, _col_spec] + [_w_spec] + [_v_spec] * 3,
    out_specs=_row_spec,
    out_shape=jax.ShapeDtypeStruct((N, D), jnp.float32),
    name="tc_gcn_layer",
)

_tc_e2d = pl.pallas_call(
    _tc_e2d_body,
    grid=(GRID,),
    in_specs=[_pp_spec, _row_spec] + [_col_spec] * 2 + [_w_spec] + [_v_spec] * 3 + [_w_spec],
    out_specs=_row_spec,
    out_shape=jax.ShapeDtypeStruct((N, D), jnp.float32),
    name="tc_layer_e2d",
)

_tc_loss = pl.pallas_call(
    _tc_loss_body,
    grid=(GRID,),
    in_specs=[_pp_spec] + [_row_spec] + [_col_spec] * 2 + [_row_spec] + [_w_spec] + [_v_spec],
    out_specs=pl.BlockSpec((1, D), lambda i: (0, 0)),
    out_shape=jax.ShapeDtypeStruct((1, D), jnp.float32),
    name="tc_recon_loss",
)


def _pad_const(total, n_real, base):
    return (base + np.arange(total - n_real) % PAD_ROWS).astype(np.int32)


_SRC_PAD = _pad_const(EPAD, E, 0)
_DST_PAD = _pad_const(EPAD, E, N)
_MK_PAD = _pad_const(MKW * EWIN, 7500, N)
_TK_PAD = _pad_const(TKW * EWIN, 6750, N)
_NN_PAD = _pad_const(NZW * EWIN, 750, N)
_NS_PAD = _pad_const(NZW * EWIN, 750, 0)


def _pad_idx_windows(a, pad):
    w = (a.shape[0] + pad.shape[0]) // EWIN
    return jnp.concatenate([a.astype(jnp.int32), pad]).reshape(w, EWIN)


@jax.jit
def kernel(x, edge_index, mask_nodes, token_nodes, noise_nodes, noise_src,
           W1, b1, g1, be1, W2, b2, g2, be2, mask_token, We2d, Wd, bd):
    num_mask = mask_nodes.shape[0]
    srcw = _pad_idx_windows(edge_index[0], _SRC_PAD)
    dstw = _pad_idx_windows(edge_index[1], _DST_PAD)
    mknw = _pad_idx_windows(mask_nodes, _MK_PAD)
    tknw = _pad_idx_windows(token_nodes, _TK_PAD)
    nnw = _pad_idx_windows(noise_nodes, _NN_PAD)
    nsw = _pad_idx_windows(noise_src, _NS_PAD)

    deg_p, kd_p, tok_p, repl_p = _sc_setup(x, dstw, mknw, tknw, nnw, nsw)

    degc = (deg_p[0, :N] + deg_p[1, :N])[:, None]
    kdc = (kd_p[0, :N] + kd_p[1, :N])[:, None]
    tokc = (tok_p[0, :N] + tok_p[1, :N])[:, None]

    hs0 = _tc_mask(x, repl_p, degc, kdc, tokc, mask_token.reshape(1, D))

    a = _sc_agg(hs0, srcw, dstw)
    hs1 = _tc_layer(a, hs0, degc,
                    W1, b1.reshape(1, D), g1.reshape(1, D), be1.reshape(1, D))

    a = _sc_agg(hs1, srcw, dstw)
    hs2 = _tc_e2d(a, hs1, degc, kdc,
                  W2, b2.reshape(1, D), g2.reshape(1, D), be2.reshape(1, D), We2d)

    a = _sc_agg(hs2, srcw, dstw)
    parts = _tc_loss(a, hs2, degc, kdc, x, Wd, bd.reshape(1, D))
    return parts[0, 0] / num_mask


# speedup vs baseline: 16.3248x; 1.0185x over previous
import functools

import jax
import jax.numpy as jnp
import numpy as np
from jax import lax
from jax.experimental import pallas as pl
from jax.experimental.pallas import tpu as pltpu
from jax.experimental.pallas import tpu_sc as plsc

N = 10000
E = 320000
D = 128
EPS = 1e-5

NPAD = 10240
PAD_ROWS = NPAD - N
EWIN = 128
EPAD = 327680
EW = EPAD // EWIN
NC = 2
NS = 16
NW = NC * NS
EW_PER_W = EW // NW
ROWS_PER_TILE = NPAD // NS
MKW = 8192 // EWIN
TKW = 8192 // EWIN
NZW = 4096 // EWIN
NB = 2
IDXC = 16

_mesh = plsc.VectorSubcoreMesh(core_axis_name="c", subcore_axis_name="s",
                               num_cores=NC, num_subcores=NS)


def _fill_const(ref, val):
    v = jnp.full((16,), val, jnp.float32)
    for j in range(EWIN // 16):
        ref[pl.ds(j * 16, 16)] = v


def _fill_rows_zero(ref):
    z = jnp.zeros((16,), jnp.float32)

    def body(r, _):
        for j in range(D // 16):
            ref[r, pl.ds(j * 16, 16)] = z
        return 0

    lax.fori_loop(0, EWIN, body, 0)


def _sc_setup_body(x_hbm, dstw, mknw, tknw, nnw, nsw,
                   deg_o, kd_o, tok_o, repl_o,
                   acc_deg, acc_kd, acc_tok, acc_repl,
                   idx_v, idx2_v, nn_v, ns_v, rows_v, ones_v, mones_v, zv_v,
                   ds0, ds1, ds2, ds3):
    c = lax.axis_index("c")
    s = lax.axis_index("s")
    wid = c * NS + s
    base = s * ROWS_PER_TILE

    _fill_const(ones_v, 1.0)
    _fill_const(mones_v, -1.0)
    _fill_const(zv_v, 0.0)
    _fill_rows_zero(rows_v)

    for b in range(ROWS_PER_TILE // EWIN):
        off = base + b * EWIN
        pltpu.sync_copy(zv_v, acc_deg.at[pl.ds(off, EWIN)])
        pltpu.sync_copy(zv_v, acc_kd.at[pl.ds(off, EWIN)])
        pltpu.sync_copy(zv_v, acc_tok.at[pl.ds(off, EWIN)])
        pltpu.sync_copy(rows_v, acc_repl.at[pl.ds(off, EWIN), :])
    plsc.subcore_barrier()

    pltpu.sync_copy(dstw.at[pl.ds(wid * EW_PER_W, EW_PER_W), :], idx_v)
    dsems = [ds0, ds1, ds2, ds3]

    def dstart(j, b):
        pltpu.async_copy(ones_v, acc_deg.at[idx_v.at[j]], dsems[b], add=True)

    def dwait(b):
        pltpu.make_async_copy(ones_v, acc_deg.at[idx_v.at[0]], dsems[b]).wait()

    for b in range(4):
        dstart(b, b)

    def deg_step(o, _):
        for b in range(4):
            j = o * 4 + b
            dwait(b)
            dstart(j + 4, b)
        return 0

    lax.fori_loop(0, EW_PER_W // 4 - 1, deg_step, 0)
    for b in range(4):
        dwait(b)

    pltpu.sync_copy(mknw.at[pl.ds(wid * (MKW // NW), MKW // NW), :], idx2_v)
    for j in range(MKW // NW):
        pltpu.sync_copy(mones_v, acc_kd.at[idx2_v.at[j]], add=True)

    pltpu.sync_copy(tknw.at[pl.ds(wid * (TKW // NW), TKW // NW), :], idx2_v)
    for j in range(TKW // NW):
        pltpu.sync_copy(ones_v, acc_tok.at[idx2_v.at[j]], add=True)

    pltpu.sync_copy(nnw.at[pl.ds(wid, 1), :], nn_v)
    pltpu.sync_copy(nsw.at[pl.ds(wid, 1), :], ns_v)
    pltpu.sync_copy(x_hbm.at[ns_v.at[0]], rows_v)
    pltpu.sync_copy(rows_v, acc_repl.at[nn_v.at[0]], add=True)

    plsc.subcore_barrier()
    pltpu.sync_copy(acc_deg.at[pl.ds(base, ROWS_PER_TILE)],
                    deg_o.at[c, pl.ds(base, ROWS_PER_TILE)])
    pltpu.sync_copy(acc_kd.at[pl.ds(base, ROWS_PER_TILE)],
                    kd_o.at[c, pl.ds(base, ROWS_PER_TILE)])
    pltpu.sync_copy(acc_tok.at[pl.ds(base, ROWS_PER_TILE)],
                    tok_o.at[c, pl.ds(base, ROWS_PER_TILE)])
    pltpu.sync_copy(acc_repl.at[pl.ds(base, ROWS_PER_TILE), :],
                    repl_o.at[c, pl.ds(base, ROWS_PER_TILE), :])


_sc_setup = pl.kernel(
    _sc_setup_body,
    out_type=(
        jax.ShapeDtypeStruct((NC, NPAD), jnp.float32),
        jax.ShapeDtypeStruct((NC, NPAD), jnp.float32),
        jax.ShapeDtypeStruct((NC, NPAD), jnp.float32),
        jax.ShapeDtypeStruct((NC, NPAD, D), jnp.float32),
    ),
    mesh=_mesh,
    scratch_types=[
        pltpu.VMEM_SHARED((NPAD,), jnp.float32),
        pltpu.VMEM_SHARED((NPAD,), jnp.float32),
        pltpu.VMEM_SHARED((NPAD,), jnp.float32),
        pltpu.VMEM_SHARED((NPAD, D), jnp.float32),
        pltpu.VMEM((EW_PER_W, EWIN), jnp.int32),
        pltpu.VMEM((MKW // NW, EWIN), jnp.int32),
        pltpu.VMEM((1, EWIN), jnp.int32),
        pltpu.VMEM((1, EWIN), jnp.int32),
        pltpu.VMEM((EWIN, D), jnp.float32),
        pltpu.VMEM((EWIN,), jnp.float32),
        pltpu.VMEM((EWIN,), jnp.float32),
        pltpu.VMEM((EWIN,), jnp.float32),
        pltpu.SemaphoreType.DMA,
        pltpu.SemaphoreType.DMA,
        pltpu.SemaphoreType.DMA,
        pltpu.SemaphoreType.DMA,
    ],
    name="sc_graph_setup",
)


def _sc_agg_body(hs_hbm, srcw, dstw, out_o, acc, src_v, dst_v, rows_v,
                 s0, s1, s2, s3):
    c = lax.axis_index("c")
    s = lax.axis_index("s")
    wid = c * NS + s
    base = s * ROWS_PER_TILE
    gsems = [s0, s1, s2, s3]

    z = jnp.zeros((16,), jnp.float32)

    def zbody(r, _):
        for j in range(D // 16):
            rows_v[0, r, pl.ds(j * 16, 16)] = z
        return 0

    lax.fori_loop(0, EWIN, zbody, 0)
    for b in range(ROWS_PER_TILE // EWIN):
        pltpu.sync_copy(rows_v.at[0], acc.at[pl.ds(base + b * EWIN, EWIN), :])
    plsc.subcore_barrier()

    ebase = wid * EW_PER_W

    def gstart(j, b):
        pltpu.async_copy(hs_hbm.at[src_v.at[j]], rows_v.at[b], gsems[b])

    def gwait(b):
        pltpu.make_async_copy(hs_hbm.at[src_v.at[0]], rows_v.at[b],
                              gsems[b]).wait()

    def chunk(o, _):
        pltpu.sync_copy(srcw.at[pl.ds(ebase + o * IDXC, IDXC), :], src_v)
        pltpu.sync_copy(dstw.at[pl.ds(ebase + o * IDXC, IDXC), :], dst_v)
        for b in range(NB):
            gstart(b, b)
        for j in range(IDXC):
            b = j % NB
            gwait(b)
            pltpu.sync_copy(rows_v.at[b], acc.at[dst_v.at[j]], add=True)
            if j + NB < IDXC:
                gstart(j + NB, b)
        return 0

    lax.fori_loop(0, EW_PER_W // IDXC, chunk, 0)

    plsc.subcore_barrier()
    pltpu.sync_copy(acc.at[pl.ds(base, ROWS_PER_TILE), :],
                    out_o.at[c, pl.ds(base, ROWS_PER_TILE), :])


_sc_agg = pl.kernel(
    _sc_agg_body,
    out_type=jax.ShapeDtypeStruct((NC, NPAD, D), jnp.float32),
    mesh=_mesh,
    scratch_types=[
        pltpu.VMEM_SHARED((NPAD, D), jnp.float32),
        pltpu.VMEM((IDXC, EWIN), jnp.int32),
        pltpu.VMEM((IDXC, EWIN), jnp.int32),
        pltpu.VMEM((NB, EWIN, D), jnp.float32),
        pltpu.SemaphoreType.DMA,
        pltpu.SemaphoreType.DMA,
        pltpu.SemaphoreType.DMA,
        pltpu.SemaphoreType.DMA,
    ],
    name="sc_gcn_agg",
)


BR = 2048
GRID = NPAD // BR
SLAB = BR // D

_row_spec = pl.BlockSpec((BR, D), lambda i: (i, 0))
_slab_spec = pl.BlockSpec((1, BR), lambda i: (0, i))
_pp_spec = pl.BlockSpec((NC, BR, D), lambda i: (0, i, 0))
_w_spec = pl.BlockSpec((D, D), lambda i: (0, 0))
_v_spec = pl.BlockSpec((1, D), lambda i: (0, 0))


def _col(slab_ref):
    return jnp.transpose(slab_ref[...], (1, 0))


def _tc_mask_body(x, rp, degs, kds, toks, mt, hs0_o):
    keep = 1.0 + _col(kds)
    dinv = lax.rsqrt(_col(degs))
    r = rp[...]
    h0 = x[...] * keep + r[0] + r[1] + _col(toks) * mt[...]
    hs0_o[...] = h0 * dinv


def _tc_layer_body(ap, hs, degs, W, b, g, be, hs_o):
    deg = _col(degs)
    dinv = lax.rsqrt(deg)
    hsv = hs[...]
    hprev = hsv * jnp.sqrt(deg)
    a = ap[...]
    agg = (a[0] + a[1] + hsv) * dinv
    t = jnp.dot(agg, W[...], preferred_element_type=jnp.float32) + b[...] + hprev
    mu = jnp.mean(t, axis=-1, keepdims=True)
    var = jnp.mean((t - mu) * (t - mu), axis=-1, keepdims=True)
    y = (t - mu) * lax.rsqrt(var + EPS) * g[...] + be[...]
    h = jnp.maximum(y, 0.0)
    hs_o[...] = h * dinv


def _tc_e2d_body(ap, hs, degs, kds, W, b, g, be, We2d, hs2_o):
    deg = _col(degs)
    dinv = lax.rsqrt(deg)
    hsv = hs[...]
    hprev = hsv * jnp.sqrt(deg)
    a = ap[...]
    agg = (a[0] + a[1] + hsv) * dinv
    t = jnp.dot(agg, W[...], preferred_element_type=jnp.float32) + b[...] + hprev
    mu = jnp.mean(t, axis=-1, keepdims=True)
    var = jnp.mean((t - mu) * (t - mu), axis=-1, keepdims=True)
    y = (t - mu) * lax.rsqrt(var + EPS) * g[...] + be[...]
    h2 = jnp.maximum(y, 0.0)
    rep = jnp.dot(h2, We2d[...], preferred_element_type=jnp.float32) * (1.0 + _col(kds))
    hs2_o[...] = rep * dinv


def _tc_loss_body(ap, hs2, degs, kds, x, Wd, bd, out):
    dinv = lax.rsqrt(_col(degs))
    a = ap[...]
    agg = (a[0] + a[1] + hs2[...]) * dinv
    recon = jnp.dot(agg, Wd[...], preferred_element_type=jnp.float32) + bd[...]
    xv = x[...]
    xn = xv / (jnp.sqrt(jnp.sum(xv * xv, axis=-1, keepdims=True)) + 1e-8)
    yn = recon / (jnp.sqrt(jnp.sum(recon * recon, axis=-1, keepdims=True)) + 1e-8)
    cos = jnp.sum(xn * yn, axis=-1, keepdims=True)
    d = 1.0 - cos
    li = d * d
    pid = pl.program_id(0)

    @pl.when(pid == 0)
    def _():
        out[...] = jnp.zeros((1, D), jnp.float32)

    rid = lax.broadcasted_iota(jnp.int32, (BR, 1), 0) + pid * BR
    w = jnp.where(rid < N, -_col(kds), 0.0)
    out[...] += jnp.broadcast_to(jnp.sum(li * w), (1, D))


_tc_mask = pl.pallas_call(
    _tc_mask_body,
    grid=(GRID,),
    in_specs=[_row_spec, _pp_spec] + [_slab_spec] * 3 + [_v_spec],
    out_specs=_row_spec,
    out_shape=jax.ShapeDtypeStruct((NPAD, D), jnp.float32),
    name="tc_mask",
)

_tc_layer = pl.pallas_call(
    _tc_layer_body,
    grid=(GRID,),
    in_specs=[_pp_spec, _row_spec, _slab_spec] + [_w_spec] + [_v_spec] * 3,
    out_specs=_row_spec,
    out_shape=jax.ShapeDtypeStruct((NPAD, D), jnp.float32),
    name="tc_gcn_layer",
)

_tc_e2d = pl.pallas_call(
    _tc_e2d_body,
    grid=(GRID,),
    in_specs=[_pp_spec, _row_spec] + [_slab_spec] * 2 + [_w_spec] + [_v_spec] * 3 + [_w_spec],
    out_specs=_row_spec,
    out_shape=jax.ShapeDtypeStruct((NPAD, D), jnp.float32),
    name="tc_layer_e2d",
)

_tc_loss = pl.pallas_call(
    _tc_loss_body,
    grid=(GRID,),
    in_specs=[_pp_spec] + [_row_spec] + [_slab_spec] * 2 + [_row_spec] + [_w_spec] + [_v_spec],
    out_specs=pl.BlockSpec((1, D), lambda i: (0, 0)),
    out_shape=jax.ShapeDtypeStruct((1, D), jnp.float32),
    name="tc_recon_loss",
)


def _pad_const(total, n_real, base):
    return (base + np.arange(total - n_real) % PAD_ROWS).astype(np.int32)


_SRC_PAD = _pad_const(EPAD, E, 0)
_DST_PAD = _pad_const(EPAD, E, N)
_MK_PAD = _pad_const(MKW * EWIN, 7500, N)
_TK_PAD = _pad_const(TKW * EWIN, 6750, N)
_NN_PAD = _pad_const(NZW * EWIN, 750, N)
_NS_PAD = _pad_const(NZW * EWIN, 750, 0)


def _pad_idx_windows(a, pad):
    w = (a.shape[0] + pad.shape[0]) // EWIN
    return jnp.concatenate([a.astype(jnp.int32), pad]).reshape(w, EWIN)


@jax.jit
def kernel(x, edge_index, mask_nodes, token_nodes, noise_nodes, noise_src,
           W1, b1, g1, be1, W2, b2, g2, be2, mask_token, We2d, Wd, bd):
    num_mask = mask_nodes.shape[0]
    srcw = _pad_idx_windows(edge_index[0], _SRC_PAD)
    dstw = _pad_idx_windows(edge_index[1], _DST_PAD)
    mknw = _pad_idx_windows(mask_nodes, _MK_PAD)
    tknw = _pad_idx_windows(token_nodes, _TK_PAD)
    nnw = _pad_idx_windows(noise_nodes, _NN_PAD)
    nsw = _pad_idx_windows(noise_src, _NS_PAD)

    x_p = jnp.zeros((NPAD, D), jnp.float32).at[:N].set(x)
    deg_p, kd_p, tok_p, repl_p = _sc_setup(x, dstw, mknw, tknw, nnw, nsw)

    degs = (deg_p[0] + deg_p[1] + 1.0).reshape(1, NPAD)
    kds = (kd_p[0] + kd_p[1]).reshape(1, NPAD)
    toks = (tok_p[0] + tok_p[1]).reshape(1, NPAD)

    hs0 = _tc_mask(x_p, repl_p, degs, kds, toks, mask_token.reshape(1, D))

    a = _sc_agg(hs0, srcw, dstw)
    hs1 = _tc_layer(a, hs0, degs,
                    W1, b1.reshape(1, D), g1.reshape(1, D), be1.reshape(1, D))

    a = _sc_agg(hs1, srcw, dstw)
    hs2 = _tc_e2d(a, hs1, degs, kds,
                  W2, b2.reshape(1, D), g2.reshape(1, D), be2.reshape(1, D), We2d)

    a = _sc_agg(hs2, srcw, dstw)
    parts = _tc_loss(a, hs2, degs, kds, x_p, Wd, bd.reshape(1, D))
    return parts[0, 0] / num_mask


# speedup vs baseline: 16.5170x; 1.0118x over previous
import functools

import jax
import jax.numpy as jnp
import numpy as np
from jax import lax
from jax.experimental import pallas as pl
from jax.experimental.pallas import tpu as pltpu
from jax.experimental.pallas import tpu_sc as plsc

N = 10000
E = 320000
D = 128
EPS = 1e-5

NPAD = 10240
PAD_ROWS = NPAD - N
EWIN = 128
EPAD = 327680
EW = EPAD // EWIN
NC = 2
NS = 16
NW = NC * NS
EW_PER_W = EW // NW
ROWS_PER_TILE = NPAD // NS
MKW = 8192 // EWIN
TKW = 8192 // EWIN
NZW = 4096 // EWIN
NB = 2
IDXC = 16

_mesh = plsc.VectorSubcoreMesh(core_axis_name="c", subcore_axis_name="s",
                               num_cores=NC, num_subcores=NS)


def _fill_const(ref, val):
    v = jnp.full((16,), val, jnp.float32)
    for j in range(EWIN // 16):
        ref[pl.ds(j * 16, 16)] = v


def _fill_rows_zero(ref):
    z = jnp.zeros((16,), jnp.float32)

    def body(r, _):
        for j in range(D // 16):
            ref[r, pl.ds(j * 16, 16)] = z
        return 0

    lax.fori_loop(0, EWIN, body, 0)


def _sc_setup_body(x_hbm, dstw, mknw, tknw, nnw, nsw,
                   deg_o, kd_o, tok_o, repl_o,
                   acc_deg, acc_kd, acc_tok, acc_repl,
                   idx_v, idx2_v, nn_v, ns_v, rows_v, ones_v, mones_v, zv_v,
                   ds0, ds1, ds2, ds3):
    c = lax.axis_index("c")
    s = lax.axis_index("s")
    wid = c * NS + s
    base = s * ROWS_PER_TILE

    _fill_const(ones_v, 1.0)
    _fill_const(mones_v, -1.0)
    _fill_const(zv_v, 0.0)
    _fill_rows_zero(rows_v)

    for b in range(ROWS_PER_TILE // EWIN):
        off = base + b * EWIN
        pltpu.sync_copy(zv_v, acc_deg.at[pl.ds(off, EWIN)])
        pltpu.sync_copy(zv_v, acc_kd.at[pl.ds(off, EWIN)])
        pltpu.sync_copy(zv_v, acc_tok.at[pl.ds(off, EWIN)])
        pltpu.sync_copy(rows_v, acc_repl.at[pl.ds(off, EWIN), :])
    plsc.subcore_barrier()

    pltpu.sync_copy(dstw.at[pl.ds(wid * EW_PER_W, EW_PER_W), :], idx_v)
    dsems = [ds0, ds1, ds2, ds3]

    def dstart(j, b):
        pltpu.async_copy(ones_v, acc_deg.at[idx_v.at[j]], dsems[b], add=True)

    def dwait(b):
        pltpu.make_async_copy(ones_v, acc_deg.at[idx_v.at[0]], dsems[b]).wait()

    for b in range(4):
        dstart(b, b)

    def deg_step(o, _):
        for b in range(4):
            j = o * 4 + b
            dwait(b)
            dstart(j + 4, b)
        return 0

    lax.fori_loop(0, EW_PER_W // 4 - 1, deg_step, 0)
    for b in range(4):
        dwait(b)

    pltpu.sync_copy(mknw.at[pl.ds(wid * (MKW // NW), MKW // NW), :], idx2_v)
    for j in range(MKW // NW):
        pltpu.sync_copy(mones_v, acc_kd.at[idx2_v.at[j]], add=True)

    pltpu.sync_copy(tknw.at[pl.ds(wid * (TKW // NW), TKW // NW), :], idx2_v)
    for j in range(TKW // NW):
        pltpu.sync_copy(ones_v, acc_tok.at[idx2_v.at[j]], add=True)

    pltpu.sync_copy(nnw.at[pl.ds(wid, 1), :], nn_v)
    pltpu.sync_copy(nsw.at[pl.ds(wid, 1), :], ns_v)
    pltpu.sync_copy(x_hbm.at[ns_v.at[0]], rows_v)
    pltpu.sync_copy(rows_v, acc_repl.at[nn_v.at[0]], add=True)

    plsc.subcore_barrier()
    pltpu.sync_copy(acc_deg.at[pl.ds(base, ROWS_PER_TILE)],
                    deg_o.at[c, pl.ds(base, ROWS_PER_TILE)])
    pltpu.sync_copy(acc_kd.at[pl.ds(base, ROWS_PER_TILE)],
                    kd_o.at[c, pl.ds(base, ROWS_PER_TILE)])
    pltpu.sync_copy(acc_tok.at[pl.ds(base, ROWS_PER_TILE)],
                    tok_o.at[c, pl.ds(base, ROWS_PER_TILE)])
    pltpu.sync_copy(acc_repl.at[pl.ds(base, ROWS_PER_TILE), :],
                    repl_o.at[c, pl.ds(base, ROWS_PER_TILE), :])


_sc_setup = pl.kernel(
    _sc_setup_body,
    out_type=(
        jax.ShapeDtypeStruct((NC, NPAD), jnp.float32),
        jax.ShapeDtypeStruct((NC, NPAD), jnp.float32),
        jax.ShapeDtypeStruct((NC, NPAD), jnp.float32),
        jax.ShapeDtypeStruct((NC, NPAD, D), jnp.float32),
    ),
    mesh=_mesh,
    scratch_types=[
        pltpu.VMEM_SHARED((NPAD,), jnp.float32),
        pltpu.VMEM_SHARED((NPAD,), jnp.float32),
        pltpu.VMEM_SHARED((NPAD,), jnp.float32),
        pltpu.VMEM_SHARED((NPAD, D), jnp.float32),
        pltpu.VMEM((EW_PER_W, EWIN), jnp.int32),
        pltpu.VMEM((MKW // NW, EWIN), jnp.int32),
        pltpu.VMEM((1, EWIN), jnp.int32),
        pltpu.VMEM((1, EWIN), jnp.int32),
        pltpu.VMEM((EWIN, D), jnp.float32),
        pltpu.VMEM((EWIN,), jnp.float32),
        pltpu.VMEM((EWIN,), jnp.float32),
        pltpu.VMEM((EWIN,), jnp.float32),
        pltpu.SemaphoreType.DMA,
        pltpu.SemaphoreType.DMA,
        pltpu.SemaphoreType.DMA,
        pltpu.SemaphoreType.DMA,
    ],
    name="sc_graph_setup",
)


def _sc_agg_body(hs_hbm, srcw, dstw, out_o, acc, src_v, dst_v, rows_v,
                 s0, s1, s2, s3):
    c = lax.axis_index("c")
    s = lax.axis_index("s")
    wid = c * NS + s
    base = s * ROWS_PER_TILE
    gsems = [s0, s1, s2, s3]

    z = jnp.zeros((16,), jnp.float32)

    def zbody(r, _):
        for j in range(D // 16):
            rows_v[0, r, pl.ds(j * 16, 16)] = z
        return 0

    lax.fori_loop(0, EWIN, zbody, 0)
    for b in range(ROWS_PER_TILE // EWIN):
        pltpu.sync_copy(rows_v.at[0], acc.at[pl.ds(base + b * EWIN, EWIN), :])
    plsc.subcore_barrier()

    ebase = wid * EW_PER_W

    def gstart(j, b):
        pltpu.async_copy(hs_hbm.at[src_v.at[j]], rows_v.at[b], gsems[b])

    def gwait(b):
        pltpu.make_async_copy(hs_hbm.at[src_v.at[0]], rows_v.at[b],
                              gsems[b]).wait()

    def chunk(o, _):
        pltpu.sync_copy(srcw.at[pl.ds(ebase + o * IDXC, IDXC), :], src_v)
        pltpu.sync_copy(dstw.at[pl.ds(ebase + o * IDXC, IDXC), :], dst_v)
        for b in range(NB):
            gstart(b, b)
        for j in range(IDXC):
            b = j % NB
            gwait(b)
            pltpu.sync_copy(rows_v.at[b], acc.at[dst_v.at[j]], add=True)
            if j + NB < IDXC:
                gstart(j + NB, b)
        return 0

    lax.fori_loop(0, EW_PER_W // IDXC, chunk, 0)

    plsc.subcore_barrier()
    pltpu.sync_copy(acc.at[pl.ds(base, ROWS_PER_TILE), :],
                    out_o.at[c, pl.ds(base, ROWS_PER_TILE), :])


_sc_agg = pl.kernel(
    _sc_agg_body,
    out_type=jax.ShapeDtypeStruct((NC, NPAD, D), jnp.float32),
    mesh=_mesh,
    scratch_types=[
        pltpu.VMEM_SHARED((NPAD, D), jnp.float32),
        pltpu.VMEM((IDXC, EWIN), jnp.int32),
        pltpu.VMEM((IDXC, EWIN), jnp.int32),
        pltpu.VMEM((NB, EWIN, D), jnp.float32),
        pltpu.SemaphoreType.DMA,
        pltpu.SemaphoreType.DMA,
        pltpu.SemaphoreType.DMA,
        pltpu.SemaphoreType.DMA,
    ],
    name="sc_gcn_agg",
)


BR = 2048
GRID = NPAD // BR
SLAB = BR // D

_row_spec = pl.BlockSpec((BR, D), lambda i: (i, 0))
_slab_spec = pl.BlockSpec((NC, BR), lambda i: (0, i))
_pp_spec = pl.BlockSpec((NC, BR, D), lambda i: (0, i, 0))
_w_spec = pl.BlockSpec((D, D), lambda i: (0, 0))
_v_spec = pl.BlockSpec((1, D), lambda i: (0, 0))


def _col(part_ref, add):
    v = part_ref[...]
    return jnp.transpose(v[0:1] + v[1:2] + add, (1, 0))


def _tc_mask_body(x, rp, degs, kds, toks, mt, hs0_o):
    keep = 1.0 + _col(kds, 0.0)
    dinv = lax.rsqrt(_col(degs, 1.0))
    r = rp[...]
    h0 = x[...] * keep + r[0] + r[1] + _col(toks, 0.0) * mt[...]
    hs0_o[...] = h0 * dinv


def _tc_layer_body(ap, hs, degs, W, b, g, be, hs_o):
    deg = _col(degs, 1.0)
    dinv = lax.rsqrt(deg)
    hsv = hs[...]
    hprev = hsv * jnp.sqrt(deg)
    a = ap[...]
    agg = (a[0] + a[1] + hsv) * dinv
    t = jnp.dot(agg, W[...], preferred_element_type=jnp.float32) + b[...] + hprev
    mu = jnp.mean(t, axis=-1, keepdims=True)
    var = jnp.mean((t - mu) * (t - mu), axis=-1, keepdims=True)
    y = (t - mu) * lax.rsqrt(var + EPS) * g[...] + be[...]
    h = jnp.maximum(y, 0.0)
    hs_o[...] = h * dinv


def _tc_e2d_body(ap, hs, degs, kds, W, b, g, be, We2d, hs2_o):
    deg = _col(degs, 1.0)
    dinv = lax.rsqrt(deg)
    hsv = hs[...]
    hprev = hsv * jnp.sqrt(deg)
    a = ap[...]
    agg = (a[0] + a[1] + hsv) * dinv
    t = jnp.dot(agg, W[...], preferred_element_type=jnp.float32) + b[...] + hprev
    mu = jnp.mean(t, axis=-1, keepdims=True)
    var = jnp.mean((t - mu) * (t - mu), axis=-1, keepdims=True)
    y = (t - mu) * lax.rsqrt(var + EPS) * g[...] + be[...]
    h2 = jnp.maximum(y, 0.0)
    rep = jnp.dot(h2, We2d[...], preferred_element_type=jnp.float32) * (1.0 + _col(kds, 0.0))
    hs2_o[...] = rep * dinv


def _tc_loss_body(ap, hs2, degs, kds, x, Wd, bd, out):
    dinv = lax.rsqrt(_col(degs, 1.0))
    a = ap[...]
    agg = (a[0] + a[1] + hs2[...]) * dinv
    recon = jnp.dot(agg, Wd[...], preferred_element_type=jnp.float32) + bd[...]
    xv = x[...]
    xn = xv / (jnp.sqrt(jnp.sum(xv * xv, axis=-1, keepdims=True)) + 1e-8)
    yn = recon / (jnp.sqrt(jnp.sum(recon * recon, axis=-1, keepdims=True)) + 1e-8)
    cos = jnp.sum(xn * yn, axis=-1, keepdims=True)
    d = 1.0 - cos
    li = d * d
    pid = pl.program_id(0)

    @pl.when(pid == 0)
    def _():
        out[...] = jnp.zeros((1, D), jnp.float32)

    rid = lax.broadcasted_iota(jnp.int32, (BR, 1), 0) + pid * BR
    w = jnp.where(rid < N, -_col(kds, 0.0), 0.0)
    out[...] += jnp.broadcast_to(jnp.sum(li * w), (1, D))


_tc_mask = pl.pallas_call(
    _tc_mask_body,
    grid=(GRID,),
    in_specs=[_row_spec, _pp_spec] + [_slab_spec] * 3 + [_v_spec],
    out_specs=_row_spec,
    out_shape=jax.ShapeDtypeStruct((NPAD, D), jnp.float32),
    name="tc_mask",
)

_tc_layer = pl.pallas_call(
    _tc_layer_body,
    grid=(GRID,),
    in_specs=[_pp_spec, _row_spec, _slab_spec] + [_w_spec] + [_v_spec] * 3,
    out_specs=_row_spec,
    out_shape=jax.ShapeDtypeStruct((NPAD, D), jnp.float32),
    name="tc_gcn_layer",
)

_tc_e2d = pl.pallas_call(
    _tc_e2d_body,
    grid=(GRID,),
    in_specs=[_pp_spec, _row_spec] + [_slab_spec] * 2 + [_w_spec] + [_v_spec] * 3 + [_w_spec],
    out_specs=_row_spec,
    out_shape=jax.ShapeDtypeStruct((NPAD, D), jnp.float32),
    name="tc_layer_e2d",
)

_tc_loss = pl.pallas_call(
    _tc_loss_body,
    grid=(GRID,),
    in_specs=[_pp_spec] + [_row_spec] + [_slab_spec] * 2 + [_row_spec] + [_w_spec] + [_v_spec],
    out_specs=pl.BlockSpec((1, D), lambda i: (0, 0)),
    out_shape=jax.ShapeDtypeStruct((1, D), jnp.float32),
    name="tc_recon_loss",
)


def _pad_const(total, n_real, base):
    return (base + np.arange(total - n_real) % PAD_ROWS).astype(np.int32)


_SRC_PAD = _pad_const(EPAD, E, 0)
_DST_PAD = _pad_const(EPAD, E, N)
_MK_PAD = _pad_const(MKW * EWIN, 7500, N)
_TK_PAD = _pad_const(TKW * EWIN, 6750, N)
_NN_PAD = _pad_const(NZW * EWIN, 750, N)
_NS_PAD = _pad_const(NZW * EWIN, 750, 0)


def _pad_idx_windows(a, pad):
    w = (a.shape[0] + pad.shape[0]) // EWIN
    return jnp.concatenate([a.astype(jnp.int32), pad]).reshape(w, EWIN)


@jax.jit
def kernel(x, edge_index, mask_nodes, token_nodes, noise_nodes, noise_src,
           W1, b1, g1, be1, W2, b2, g2, be2, mask_token, We2d, Wd, bd):
    num_mask = mask_nodes.shape[0]
    srcw = _pad_idx_windows(edge_index[0], _SRC_PAD)
    dstw = _pad_idx_windows(edge_index[1], _DST_PAD)
    mknw = _pad_idx_windows(mask_nodes, _MK_PAD)
    tknw = _pad_idx_windows(token_nodes, _TK_PAD)
    nnw = _pad_idx_windows(noise_nodes, _NN_PAD)
    nsw = _pad_idx_windows(noise_src, _NS_PAD)

    x_p = jnp.zeros((NPAD, D), jnp.float32).at[:N].set(x)
    deg_p, kd_p, tok_p, repl_p = _sc_setup(x, dstw, mknw, tknw, nnw, nsw)

    hs0 = _tc_mask(x_p, repl_p, deg_p, kd_p, tok_p, mask_token.reshape(1, D))

    a = _sc_agg(hs0, srcw, dstw)
    hs1 = _tc_layer(a, hs0, deg_p,
                    W1, b1.reshape(1, D), g1.reshape(1, D), be1.reshape(1, D))

    a = _sc_agg(hs1, srcw, dstw)
    hs2 = _tc_e2d(a, hs1, deg_p, kd_p,
                  W2, b2.reshape(1, D), g2.reshape(1, D), be2.reshape(1, D), We2d)

    a = _sc_agg(hs2, srcw, dstw)
    parts = _tc_loss(a, hs2, deg_p, kd_p, x_p, Wd, bd.reshape(1, D))
    return parts[0, 0] / num_mask


# speedup vs baseline: 16.5238x; 1.0004x over previous
import jax
import jax.numpy as jnp
import numpy as np
from jax import lax
from jax.experimental import pallas as pl
from jax.experimental.pallas import tpu as pltpu
from jax.experimental.pallas import tpu_sc as plsc

N = 10000
E = 320000
D = 128
EPS = 1e-5

NPAD = 10240
PAD_ROWS = NPAD - N
EWIN = 128
EPAD = 327680
EW = EPAD // EWIN
NC = 2
NS = 16
NW = NC * NS
EW_PER_W = EW // NW
ROWS_PER_TILE = NPAD // NS
MKW = 8192 // EWIN
TKW = 8192 // EWIN
NZW = 4096 // EWIN
NB = 2
IDXC = 16

_mesh = plsc.VectorSubcoreMesh(core_axis_name="c", subcore_axis_name="s",
                               num_cores=NC, num_subcores=NS)


def _fill_const(ref, val):
    v = jnp.full((16,), val, jnp.float32)
    for j in range(EWIN // 16):
        ref[pl.ds(j * 16, 16)] = v


def _fill_rows_zero(ref):
    z = jnp.zeros((16,), jnp.float32)

    def body(r, _):
        for j in range(D // 16):
            ref[r, pl.ds(j * 16, 16)] = z
        return 0

    lax.fori_loop(0, EWIN, body, 0)


def _sc_setup_body(x_hbm, dstw, mknw, tknw, nnw, nsw,
                   deg_o, kd_o, tok_o, repl_o,
                   acc_deg, acc_kd, acc_tok, acc_repl,
                   idx_v, idx2_v, nn_v, ns_v, rows_v, ones_v, mones_v, zv_v,
                   ds0, ds1, ds2, ds3):
    c = lax.axis_index("c")
    s = lax.axis_index("s")
    wid = c * NS + s
    base = s * ROWS_PER_TILE

    _fill_const(ones_v, 1.0)
    _fill_const(mones_v, -1.0)
    _fill_const(zv_v, 0.0)
    _fill_rows_zero(rows_v)

    for b in range(ROWS_PER_TILE // EWIN):
        off = base + b * EWIN
        pltpu.sync_copy(zv_v, acc_deg.at[pl.ds(off, EWIN)])
        pltpu.sync_copy(zv_v, acc_kd.at[pl.ds(off, EWIN)])
        pltpu.sync_copy(zv_v, acc_tok.at[pl.ds(off, EWIN)])
        pltpu.sync_copy(rows_v, acc_repl.at[pl.ds(off, EWIN), :])
    plsc.subcore_barrier()

    pltpu.sync_copy(dstw.at[pl.ds(wid * EW_PER_W, EW_PER_W), :], idx_v)
    dsems = [ds0, ds1, ds2, ds3]

    def dstart(j, b):
        pltpu.async_copy(ones_v, acc_deg.at[idx_v.at[j]], dsems[b], add=True)

    def dwait(b):
        pltpu.make_async_copy(ones_v, acc_deg.at[idx_v.at[0]], dsems[b]).wait()

    for b in range(4):
        dstart(b, b)

    def deg_step(o, _):
        for b in range(4):
            j = o * 4 + b
            dwait(b)
            dstart(j + 4, b)
        return 0

    lax.fori_loop(0, EW_PER_W // 4 - 1, deg_step, 0)
    for b in range(4):
        dwait(b)

    pltpu.sync_copy(mknw.at[pl.ds(wid * (MKW // NW), MKW // NW), :], idx2_v)
    for j in range(MKW // NW):
        pltpu.sync_copy(mones_v, acc_kd.at[idx2_v.at[j]], add=True)

    pltpu.sync_copy(tknw.at[pl.ds(wid * (TKW // NW), TKW // NW), :], idx2_v)
    for j in range(TKW // NW):
        pltpu.sync_copy(ones_v, acc_tok.at[idx2_v.at[j]], add=True)

    pltpu.sync_copy(nnw.at[pl.ds(wid, 1), :], nn_v)
    pltpu.sync_copy(nsw.at[pl.ds(wid, 1), :], ns_v)
    pltpu.sync_copy(x_hbm.at[ns_v.at[0]], rows_v)
    pltpu.sync_copy(rows_v, acc_repl.at[nn_v.at[0]], add=True)

    plsc.subcore_barrier()
    pltpu.sync_copy(acc_deg.at[pl.ds(base, ROWS_PER_TILE)],
                    deg_o.at[c, pl.ds(base, ROWS_PER_TILE)])
    pltpu.sync_copy(acc_kd.at[pl.ds(base, ROWS_PER_TILE)],
                    kd_o.at[c, pl.ds(base, ROWS_PER_TILE)])
    pltpu.sync_copy(acc_tok.at[pl.ds(base, ROWS_PER_TILE)],
                    tok_o.at[c, pl.ds(base, ROWS_PER_TILE)])
    pltpu.sync_copy(acc_repl.at[pl.ds(base, ROWS_PER_TILE), :],
                    repl_o.at[c, pl.ds(base, ROWS_PER_TILE), :])


_sc_setup = pl.kernel(
    _sc_setup_body,
    out_type=(
        jax.ShapeDtypeStruct((NC, NPAD), jnp.float32),
        jax.ShapeDtypeStruct((NC, NPAD), jnp.float32),
        jax.ShapeDtypeStruct((NC, NPAD), jnp.float32),
        jax.ShapeDtypeStruct((NC, NPAD, D), jnp.float32),
    ),
    mesh=_mesh,
    scratch_types=[
        pltpu.VMEM_SHARED((NPAD,), jnp.float32),
        pltpu.VMEM_SHARED((NPAD,), jnp.float32),
        pltpu.VMEM_SHARED((NPAD,), jnp.float32),
        pltpu.VMEM_SHARED((NPAD, D), jnp.float32),
        pltpu.VMEM((EW_PER_W, EWIN), jnp.int32),
        pltpu.VMEM((MKW // NW, EWIN), jnp.int32),
        pltpu.VMEM((1, EWIN), jnp.int32),
        pltpu.VMEM((1, EWIN), jnp.int32),
        pltpu.VMEM((EWIN, D), jnp.float32),
        pltpu.VMEM((EWIN,), jnp.float32),
        pltpu.VMEM((EWIN,), jnp.float32),
        pltpu.VMEM((EWIN,), jnp.float32),
        pltpu.SemaphoreType.DMA,
        pltpu.SemaphoreType.DMA,
        pltpu.SemaphoreType.DMA,
        pltpu.SemaphoreType.DMA,
    ],
    name="sc_graph_setup",
)


def _sc_agg_body(hs_hbm, srcw, dstw, out_o, acc, src_v, dst_v, rows_v,
                 s0, s1, s2, s3):
    c = lax.axis_index("c")
    s = lax.axis_index("s")
    wid = c * NS + s
    base = s * ROWS_PER_TILE
    gsems = [s0, s1, s2, s3]

    z = jnp.zeros((16,), jnp.float32)

    def zbody(r, _):
        for j in range(D // 16):
            rows_v[0, r, pl.ds(j * 16, 16)] = z
        return 0

    lax.fori_loop(0, EWIN, zbody, 0)
    for b in range(ROWS_PER_TILE // EWIN):
        pltpu.sync_copy(rows_v.at[0], acc.at[pl.ds(base + b * EWIN, EWIN), :])
    plsc.subcore_barrier()

    ebase = wid * EW_PER_W

    def gstart(j, b):
        pltpu.async_copy(hs_hbm.at[src_v.at[j]], rows_v.at[b], gsems[b])

    def gwait(b):
        pltpu.make_async_copy(hs_hbm.at[src_v.at[0]], rows_v.at[b],
                              gsems[b]).wait()

    def chunk(o, _):
        pltpu.sync_copy(srcw.at[pl.ds(ebase + o * IDXC, IDXC), :], src_v)
        pltpu.sync_copy(dstw.at[pl.ds(ebase + o * IDXC, IDXC), :], dst_v)
        for b in range(NB):
            gstart(b, b)
        for j in range(IDXC):
            b = j % NB
            gwait(b)
            pltpu.sync_copy(rows_v.at[b], acc.at[dst_v.at[j]], add=True)
            if j + NB < IDXC:
                gstart(j + NB, b)
        return 0

    lax.fori_loop(0, EW_PER_W // IDXC, chunk, 0)

    plsc.subcore_barrier()
    pltpu.sync_copy(acc.at[pl.ds(base, ROWS_PER_TILE), :],
                    out_o.at[c, pl.ds(base, ROWS_PER_TILE), :])


_sc_agg = pl.kernel(
    _sc_agg_body,
    out_type=jax.ShapeDtypeStruct((NC, NPAD, D), jnp.float32),
    mesh=_mesh,
    scratch_types=[
        pltpu.VMEM_SHARED((NPAD, D), jnp.float32),
        pltpu.VMEM((IDXC, EWIN), jnp.int32),
        pltpu.VMEM((IDXC, EWIN), jnp.int32),
        pltpu.VMEM((NB, EWIN, D), jnp.float32),
        pltpu.SemaphoreType.DMA,
        pltpu.SemaphoreType.DMA,
        pltpu.SemaphoreType.DMA,
        pltpu.SemaphoreType.DMA,
    ],
    name="sc_gcn_agg",
)


BR = 2048
GRID = NPAD // BR
SLAB = BR // D

_row_spec = pl.BlockSpec((BR, D), lambda i: (i, 0))
_slab_spec = pl.BlockSpec((NC, BR), lambda i: (0, i))
_pp_spec = pl.BlockSpec((NC, BR, D), lambda i: (0, i, 0))
_w_spec = pl.BlockSpec((D, D), lambda i: (0, 0))
_v_spec = pl.BlockSpec((1, D), lambda i: (0, 0))


def _col(part_ref, add):
    v = part_ref[...]
    return jnp.transpose(v[0:1] + v[1:2] + add, (1, 0))


def _tc_mask_body(x, rp, degs, kds, toks, mt, hs0_o):
    keep = 1.0 + _col(kds, 0.0)
    dinv = lax.rsqrt(_col(degs, 1.0))
    r = rp[...]
    h0 = x[...] * keep + r[0] + r[1] + _col(toks, 0.0) * mt[...]
    hs0_o[...] = h0 * dinv


def _tc_layer_body(ap, hs, degs, W, b, g, be, hs_o):
    deg = _col(degs, 1.0)
    dinv = lax.rsqrt(deg)
    hsv = hs[...]
    hprev = hsv * jnp.sqrt(deg)
    a = ap[...]
    agg = (a[0] + a[1] + hsv) * dinv
    t = jnp.dot(agg, W[...], preferred_element_type=jnp.float32) + b[...] + hprev
    mu = jnp.mean(t, axis=-1, keepdims=True)
    var = jnp.mean((t - mu) * (t - mu), axis=-1, keepdims=True)
    y = (t - mu) * lax.rsqrt(var + EPS) * g[...] + be[...]
    h = jnp.maximum(y, 0.0)
    hs_o[...] = h * dinv


def _tc_e2d_body(ap, hs, degs, kds, W, b, g, be, We2d, hs2_o):
    deg = _col(degs, 1.0)
    dinv = lax.rsqrt(deg)
    hsv = hs[...]
    hprev = hsv * jnp.sqrt(deg)
    a = ap[...]
    agg = (a[0] + a[1] + hsv) * dinv
    t = jnp.dot(agg, W[...], preferred_element_type=jnp.float32) + b[...] + hprev
    mu = jnp.mean(t, axis=-1, keepdims=True)
    var = jnp.mean((t - mu) * (t - mu), axis=-1, keepdims=True)
    y = (t - mu) * lax.rsqrt(var + EPS) * g[...] + be[...]
    h2 = jnp.maximum(y, 0.0)
    rep = jnp.dot(h2, We2d[...], preferred_element_type=jnp.float32) * (1.0 + _col(kds, 0.0))
    hs2_o[...] = rep * dinv


def _tc_loss_body(ap, hs2, degs, kds, x, Wd, bd, out):
    dinv = lax.rsqrt(_col(degs, 1.0))
    a = ap[...]
    agg = (a[0] + a[1] + hs2[...]) * dinv
    recon = jnp.dot(agg, Wd[...], preferred_element_type=jnp.float32) + bd[...]
    xv = x[...]
    xn = xv / (jnp.sqrt(jnp.sum(xv * xv, axis=-1, keepdims=True)) + 1e-8)
    yn = recon / (jnp.sqrt(jnp.sum(recon * recon, axis=-1, keepdims=True)) + 1e-8)
    cos = jnp.sum(xn * yn, axis=-1, keepdims=True)
    d = 1.0 - cos
    li = d * d
    pid = pl.program_id(0)

    @pl.when(pid == 0)
    def _():
        out[...] = jnp.zeros((1, D), jnp.float32)

    rid = lax.broadcasted_iota(jnp.int32, (BR, 1), 0) + pid * BR
    w = jnp.where(rid < N, -_col(kds, 0.0), 0.0)
    out[...] += jnp.broadcast_to(jnp.sum(li * w), (1, D))


_tc_mask = pl.pallas_call(
    _tc_mask_body,
    grid=(GRID,),
    in_specs=[_row_spec, _pp_spec] + [_slab_spec] * 3 + [_v_spec],
    out_specs=_row_spec,
    out_shape=jax.ShapeDtypeStruct((NPAD, D), jnp.float32),
    name="tc_mask",
)

_tc_layer = pl.pallas_call(
    _tc_layer_body,
    grid=(GRID,),
    in_specs=[_pp_spec, _row_spec, _slab_spec] + [_w_spec] + [_v_spec] * 3,
    out_specs=_row_spec,
    out_shape=jax.ShapeDtypeStruct((NPAD, D), jnp.float32),
    name="tc_gcn_layer",
)

_tc_e2d = pl.pallas_call(
    _tc_e2d_body,
    grid=(GRID,),
    in_specs=[_pp_spec, _row_spec] + [_slab_spec] * 2 + [_w_spec] + [_v_spec] * 3 + [_w_spec],
    out_specs=_row_spec,
    out_shape=jax.ShapeDtypeStruct((NPAD, D), jnp.float32),
    name="tc_layer_e2d",
)

_tc_loss = pl.pallas_call(
    _tc_loss_body,
    grid=(GRID,),
    in_specs=[_pp_spec] + [_row_spec] + [_slab_spec] * 2 + [_row_spec] + [_w_spec] + [_v_spec],
    out_specs=pl.BlockSpec((1, D), lambda i: (0, 0)),
    out_shape=jax.ShapeDtypeStruct((1, D), jnp.float32),
    name="tc_recon_loss",
)


def _pad_const(total, n_real, base):
    return (base + np.arange(total - n_real) % PAD_ROWS).astype(np.int32)


_SRC_PAD = _pad_const(EPAD, E, 0)
_DST_PAD = _pad_const(EPAD, E, N)
_MK_PAD = _pad_const(MKW * EWIN, 7500, N)
_TK_PAD = _pad_const(TKW * EWIN, 6750, N)
_NN_PAD = _pad_const(NZW * EWIN, 750, N)
_NS_PAD = _pad_const(NZW * EWIN, 750, 0)


def _pad_idx_windows(a, pad):
    w = (a.shape[0] + pad.shape[0]) // EWIN
    return jnp.concatenate([a.astype(jnp.int32), pad]).reshape(w, EWIN)


@jax.jit
def kernel(x, edge_index, mask_nodes, token_nodes, noise_nodes, noise_src,
           W1, b1, g1, be1, W2, b2, g2, be2, mask_token, We2d, Wd, bd):
    num_mask = mask_nodes.shape[0]
    srcw = _pad_idx_windows(edge_index[0], _SRC_PAD)
    dstw = _pad_idx_windows(edge_index[1], _DST_PAD)
    mknw = _pad_idx_windows(mask_nodes, _MK_PAD)
    tknw = _pad_idx_windows(token_nodes, _TK_PAD)
    nnw = _pad_idx_windows(noise_nodes, _NN_PAD)
    nsw = _pad_idx_windows(noise_src, _NS_PAD)

    x_p = jnp.zeros((NPAD, D), jnp.float32).at[:N].set(x)
    deg_p, kd_p, tok_p, repl_p = _sc_setup(x, dstw, mknw, tknw, nnw, nsw)

    hs0 = _tc_mask(x_p, repl_p, deg_p, kd_p, tok_p, mask_token.reshape(1, D))

    a = _sc_agg(hs0, srcw, dstw)
    hs1 = _tc_layer(a, hs0, deg_p,
                    W1, b1.reshape(1, D), g1.reshape(1, D), be1.reshape(1, D))

    a = _sc_agg(hs1, srcw, dstw)
    hs2 = _tc_e2d(a, hs1, deg_p, kd_p,
                  W2, b2.reshape(1, D), g2.reshape(1, D), be2.reshape(1, D), We2d)

    a = _sc_agg(hs2, srcw, dstw)
    parts = _tc_loss(a, hs2, deg_p, kd_p, x_p, Wd, bd.reshape(1, D))
    return parts[0, 0] / num_mask
